# baseline (speedup 1.0000x reference)
import functools

import jax
import jax.numpy as jnp
from jax import lax
from jax.experimental import pallas as pl
from jax.experimental.pallas import tpu as pltpu
from jax.experimental.pallas import tpu_sc as plsc

N = 10000
D = 128
NC = 2
NS = 16
NW = NC * NS
CH = 128
NCHS = 160
TOTC = NS * NCHS
E_PAD = TOTC * CH
NCHUNK = 80
EPT = NCHUNK * CH
NPAD = 10240
RPT = NPAD // NS
RCH = RPT // CH


@functools.cache
def _build_sc_kernels():
    mesh = plsc.VectorSubcoreMesh(core_axis_name="c", subcore_axis_name="s",
                                  num_cores=NC, num_subcores=NS)

    @functools.partial(
        pl.kernel,
        out_type=jax.ShapeDtypeStruct((NW, NPAD), jnp.float32),
        mesh=mesh,
        scratch_types=[
            pltpu.VMEM((EPT,), jnp.int32),
            pltpu.VMEM((NPAD,), jnp.float32),
        ],
        compiler_params=pltpu.CompilerParams(needs_layout_passes=False),
    )
    def sc_degree(dst2, hist_out, didx, hist_v):
        wid = lax.axis_index("c") * NS + lax.axis_index("s")
        pltpu.sync_copy(dst2.at[wid], didx)

        def zero(i, _):
            hist_v[pl.ds(i * 16, 16)] = jnp.zeros((16,), jnp.float32)
            return _

        lax.fori_loop(0, NPAD // 16, zero, None)

        ones = jnp.ones((16,), jnp.float32)

        def body(i, _):
            idx = didx[pl.ds(i * 16, 16)]
            plsc.addupdate_scatter(hist_v, [idx], ones)
            return _

        lax.fori_loop(0, EPT // 16, body, None)
        pltpu.sync_copy(hist_v, hist_out.at[wid])

    NBUF = 2
    IRING = 4

    mesh1 = plsc.VectorSubcoreMesh(core_axis_name="c", subcore_axis_name="s",
                                   num_cores=1, num_subcores=NS)

    @functools.partial(
        pl.kernel,
        out_type=jax.ShapeDtypeStruct((NPAD, D), jnp.float32),
        mesh=mesh1,
        scratch_types=[
            pltpu.VMEM((IRING, 2, CH), jnp.int32),
            [pltpu.VMEM((CH, D), jnp.float32)] * NBUF,
            [pltpu.SemaphoreType.DMA] * NBUF,
            [pltpu.SemaphoreType.DMA] * IRING,
            pltpu.VMEM_SHARED((NPAD, D), jnp.float32),
        ],
    )
    def sc_scatter(ed3, y, zeros, parts, ring, bufs, gsems, isems, agg):
        sid = lax.axis_index("s")

        pltpu.sync_copy(zeros, bufs[0])
        for k in range(RCH):
            pltpu.sync_copy(bufs[0], agg.at[pl.ds(sid * RPT + k * CH, CH)])
        plsc.subcore_barrier()

        def run(base, nch):
            for c in range(NBUF):
                pltpu.async_copy(ed3.at[base + c], ring.at[c], isems[c])

            def body(i, _):
                for j in range(IRING):
                    c = i * IRING + j
                    b = j % NBUF
                    pltpu.make_async_copy(ed3.at[base + c], ring.at[j],
                                          isems[j]).wait()
                    pltpu.async_copy(y.at[ring.at[j, 0]], bufs[b], gsems[b])

                    def _drain():
                        jp = (j - 1) % IRING
                        pltpu.make_async_copy(y.at[ring.at[jp, 0]],
                                              bufs[1 - b],
                                              gsems[1 - b]).wait()
                        pltpu.sync_copy(bufs[1 - b], agg.at[ring.at[jp, 1]],
                                        add=True)

                    if j == 0:
                        pl.when(i > 0)(_drain)
                    else:
                        _drain()

                    def _prefetch():
                        pltpu.async_copy(ed3.at[base + c + 2],
                                         ring.at[(j + 2) % IRING],
                                         isems[(j + 2) % IRING])

                    if j < 2:
                        _prefetch()
                    else:
                        pl.when(i < nch // IRING - 1)(_prefetch)
                return _

            lax.fori_loop(0, nch // IRING, body, None)
            pltpu.make_async_copy(y.at[ring.at[IRING - 1, 0]], bufs[1],
                                  gsems[1]).wait()
            pltpu.sync_copy(bufs[1], agg.at[ring.at[IRING - 1, 1]], add=True)

        run(sid * NCHS, NCHS)
        plsc.subcore_barrier()

        for k in range(RCH):
            row = sid * RPT + k * CH
            pltpu.sync_copy(agg.at[pl.ds(row, CH)], bufs[0])
            pltpu.sync_copy(bufs[0], parts.at[pl.ds(row, CH)])

    return sc_degree, sc_scatter


_BLK = 1024
_GRID = NPAD // _BLK


def _dinv_of(hist_blk):
    deg = jnp.sum(hist_blk, axis=0) + 1.0
    return lax.rsqrt(deg)


def _tc_y1_body(hist_ref, x_ref, w_ref, y_ref):
    dinv = _dinv_of(hist_ref[...])
    xw = jnp.dot(x_ref[...], w_ref[...],
                 preferred_element_type=jnp.float32,
                 precision=lax.Precision.HIGHEST)
    y_ref[...] = xw * dinv[:, None]


def _tc_layer1_body(hist_ref, parts_ref, y1_ref, b1_ref, w2_ref,
                    z_ref, y2_ref):
    dinv = _dinv_of(hist_ref[...])
    agg = parts_ref[...] + y1_ref[...]
    z = jnp.maximum(agg * dinv[:, None] + b1_ref[...], 0.0)
    z_ref[...] = z
    zw = jnp.dot(z, w2_ref[...],
                 preferred_element_type=jnp.float32,
                 precision=lax.Precision.HIGHEST)
    y2_ref[...] = zw * dinv[:, None]


def _tc_layer2_body(hist_ref, parts_ref, y2_ref, b2_ref, out_ref):
    dinv = _dinv_of(hist_ref[...])
    h = (parts_ref[...] + y2_ref[...]) * dinv[:, None]
    h = h + b2_ref[...]
    m = jnp.max(h, axis=1, keepdims=True)
    e = jnp.exp(h - m)
    out_ref[...] = e / jnp.sum(e, axis=1, keepdims=True)


_hist_spec = pl.BlockSpec((NW, _BLK), lambda j: (0, j))
_row_spec = pl.BlockSpec((_BLK, D), lambda j: (j, 0))
_parts_spec = pl.BlockSpec((_BLK, D), lambda j: (j, 0))
_w_spec = pl.BlockSpec((D, D), lambda j: (0, 0))
_b_spec = pl.BlockSpec((1, D), lambda j: (0, 0))
_rows_out = jax.ShapeDtypeStruct((NPAD, D), jnp.float32)

_tc_y1 = pl.pallas_call(
    _tc_y1_body, grid=(_GRID,),
    in_specs=[_hist_spec, _row_spec, _w_spec],
    out_specs=_row_spec, out_shape=_rows_out)

_tc_layer1 = pl.pallas_call(
    _tc_layer1_body, grid=(_GRID,),
    in_specs=[_hist_spec, _parts_spec, _row_spec, _b_spec, _w_spec],
    out_specs=(_row_spec, _row_spec), out_shape=(_rows_out, _rows_out))

_tc_layer2 = pl.pallas_call(
    _tc_layer2_body, grid=(_GRID,),
    in_specs=[_hist_spec, _parts_spec, _row_spec, _b_spec],
    out_specs=_row_spec, out_shape=_rows_out)


def kernel(x, edge_index, W1, b1, W2, b2):
    src = edge_index[0]
    dst = edge_index[1]
    npad_e = E_PAD - src.shape[0]
    srcp = jnp.concatenate([src, jnp.zeros((npad_e,), jnp.int32)])
    dstp = jnp.concatenate([dst, jnp.full((npad_e,), N, jnp.int32)])
    ed3 = jnp.stack([srcp.reshape(TOTC, CH), dstp.reshape(TOTC, CH)], axis=1)
    dst2 = dstp.reshape(NW, EPT)
    xp = jnp.pad(x, ((0, NPAD - N), (0, 0)))
    zeros = jnp.zeros((CH, D), jnp.float32)
    b1r = b1.reshape(1, D)
    b2r = b2.reshape(1, D)

    sc_degree, sc_scatter = _build_sc_kernels()
    hist = sc_degree(dst2)
    y1 = _tc_y1(hist, xp, W1)
    parts1 = sc_scatter(ed3, y1, zeros)
    z, y2 = _tc_layer1(hist, parts1, y1, b1r, W2)
    parts2 = sc_scatter(ed3, y2, zeros)
    out = _tc_layer2(hist, parts2, y2, b2r)
    return (z[:N], out[:N])


# speedup vs baseline: 1.1897x; 1.1897x over previous
import functools

import jax
import jax.numpy as jnp
from jax import lax
from jax.experimental import pallas as pl
from jax.experimental.pallas import tpu as pltpu
from jax.experimental.pallas import tpu_sc as plsc

N = 10000
D = 128
NC = 2
NS = 16
NW = NC * NS
CH = 64
NCH0 = 256
NCH1 = 64
TOTC = NS * (NCH0 + NCH1)
E_PAD = TOTC * CH
EPT = E_PAD // NW
NPAD = 10240
RPT = NPAD // NS
RCH = RPT // CH


@functools.cache
def _build_sc_kernels():
    mesh = plsc.VectorSubcoreMesh(core_axis_name="c", subcore_axis_name="s",
                                  num_cores=NC, num_subcores=NS)

    @functools.partial(
        pl.kernel,
        out_type=jax.ShapeDtypeStruct((NW, NPAD), jnp.float32),
        mesh=mesh,
        scratch_types=[
            pltpu.VMEM((EPT,), jnp.int32),
            pltpu.VMEM((NPAD,), jnp.float32),
        ],
        compiler_params=pltpu.CompilerParams(needs_layout_passes=False),
    )
    def sc_degree(dst2, hist_out, didx, hist_v):
        wid = lax.axis_index("c") * NS + lax.axis_index("s")
        pltpu.sync_copy(dst2.at[wid], didx)

        def zero(i, _):
            hist_v[pl.ds(i * 16, 16)] = jnp.zeros((16,), jnp.float32)
            return _

        lax.fori_loop(0, NPAD // 16, zero, None)

        ones = jnp.ones((16,), jnp.float32)

        def body(i, _):
            idx = didx[pl.ds(i * 16, 16)]
            plsc.addupdate_scatter(hist_v, [idx], ones)
            return _

        lax.fori_loop(0, EPT // 16, body, None)
        pltpu.sync_copy(hist_v, hist_out.at[wid])

    NBUF = 4
    IRING = 8

    @functools.partial(
        pl.kernel,
        out_type=jax.ShapeDtypeStruct((NC, NPAD, D), jnp.float32),
        mesh=mesh,
        scratch_types=[
            pltpu.VMEM((IRING, 2, CH), jnp.int32),
            [pltpu.VMEM((CH, D), jnp.float32)] * NBUF,
            [pltpu.SemaphoreType.DMA] * NBUF,
            [pltpu.SemaphoreType.DMA] * IRING,
            pltpu.VMEM_SHARED((NPAD, D), jnp.float32),
        ],
    )
    def sc_scatter(ed3, y, zeros, parts, ring, bufs, gsems, isems, agg):
        cid = lax.axis_index("c")
        sid = lax.axis_index("s")

        pltpu.sync_copy(zeros, bufs[0])
        for k in range(RCH):
            pltpu.sync_copy(bufs[0], agg.at[pl.ds(sid * RPT + k * CH, CH)])
        plsc.subcore_barrier()

        def drain(slot, bd):
            pltpu.make_async_copy(y.at[ring.at[slot, 0]], bufs[bd],
                                  gsems[bd]).wait()
            pltpu.sync_copy(bufs[bd], agg.at[ring.at[slot, 1]], add=True)

        def run(base, nch):
            for c in range(NBUF):
                pltpu.async_copy(ed3.at[base + c], ring.at[c], isems[c])

            def body(i, _):
                for j in range(IRING):
                    c = i * IRING + j
                    b = j % NBUF
                    pltpu.make_async_copy(ed3.at[base + c], ring.at[j],
                                          isems[j]).wait()
                    pltpu.async_copy(y.at[ring.at[j, 0]], bufs[b], gsems[b])

                    if j < 3:
                        pl.when(i > 0)(
                            lambda jd=(j - 3) % IRING, bd=(j + 1) % NBUF:
                            drain(jd, bd))
                    else:
                        drain((j - 3) % IRING, (j + 1) % NBUF)

                    def _prefetch():
                        pltpu.async_copy(ed3.at[base + c + NBUF],
                                         ring.at[(j + NBUF) % IRING],
                                         isems[(j + NBUF) % IRING])

                    if j < NBUF:
                        _prefetch()
                    else:
                        pl.when(i < nch // IRING - 1)(_prefetch)
                return _

            lax.fori_loop(0, nch // IRING, body, None)
            for t in range(3):
                jj = IRING - 3 + t
                drain(jj, jj % NBUF)

        @pl.when(cid == 0)
        def _core0():
            run(sid * NCH0, NCH0)

        @pl.when(cid == 1)
        def _core1():
            run(NS * NCH0 + sid * NCH1, NCH1)

        plsc.subcore_barrier()

        for k in range(RCH):
            row = sid * RPT + k * CH
            pltpu.sync_copy(agg.at[pl.ds(row, CH)], bufs[0])
            pltpu.sync_copy(bufs[0], parts.at[cid, pl.ds(row, CH)])

    return sc_degree, sc_scatter


_BLK = 1024
_GRID = NPAD // _BLK


def _dinv_of(hist_blk):
    deg = jnp.sum(hist_blk, axis=0) + 1.0
    return lax.rsqrt(deg)


def _tc_y1_body(hist_ref, x_ref, w_ref, y_ref):
    dinv = _dinv_of(hist_ref[...])
    xw = jnp.dot(x_ref[...], w_ref[...],
                 preferred_element_type=jnp.float32,
                 precision=lax.Precision.HIGHEST)
    y_ref[...] = xw * dinv[:, None]


def _tc_layer1_body(hist_ref, parts_ref, y1_ref, b1_ref, w2_ref,
                    z_ref, y2_ref):
    dinv = _dinv_of(hist_ref[...])
    agg = jnp.sum(parts_ref[...], axis=0) + y1_ref[...]
    z = jnp.maximum(agg * dinv[:, None] + b1_ref[...], 0.0)
    z_ref[...] = z
    zw = jnp.dot(z, w2_ref[...],
                 preferred_element_type=jnp.float32,
                 precision=lax.Precision.HIGHEST)
    y2_ref[...] = zw * dinv[:, None]


def _tc_layer2_body(hist_ref, parts_ref, y2_ref, b2_ref, out_ref):
    dinv = _dinv_of(hist_ref[...])
    h = (jnp.sum(parts_ref[...], axis=0) + y2_ref[...]) * dinv[:, None]
    h = h + b2_ref[...]
    m = jnp.max(h, axis=1, keepdims=True)
    e = jnp.exp(h - m)
    out_ref[...] = e / jnp.sum(e, axis=1, keepdims=True)


_hist_spec = pl.BlockSpec((NW, _BLK), lambda j: (0, j))
_row_spec = pl.BlockSpec((_BLK, D), lambda j: (j, 0))
_parts_spec = pl.BlockSpec((NC, _BLK, D), lambda j: (0, j, 0))
_w_spec = pl.BlockSpec((D, D), lambda j: (0, 0))
_b_spec = pl.BlockSpec((1, D), lambda j: (0, 0))
_rows_out = jax.ShapeDtypeStruct((NPAD, D), jnp.float32)

_tc_y1 = pl.pallas_call(
    _tc_y1_body, grid=(_GRID,),
    in_specs=[_hist_spec, _row_spec, _w_spec],
    out_specs=_row_spec, out_shape=_rows_out)

_tc_layer1 = pl.pallas_call(
    _tc_layer1_body, grid=(_GRID,),
    in_specs=[_hist_spec, _parts_spec, _row_spec, _b_spec, _w_spec],
    out_specs=(_row_spec, _row_spec), out_shape=(_rows_out, _rows_out))

_tc_layer2 = pl.pallas_call(
    _tc_layer2_body, grid=(_GRID,),
    in_specs=[_hist_spec, _parts_spec, _row_spec, _b_spec],
    out_specs=_row_spec, out_shape=_rows_out)


def kernel(x, edge_index, W1, b1, W2, b2):
    src = edge_index[0]
    dst = edge_index[1]
    npad_e = E_PAD - src.shape[0]
    srcp = jnp.concatenate([src, jnp.zeros((npad_e,), jnp.int32)])
    dstp = jnp.concatenate([dst, jnp.full((npad_e,), N, jnp.int32)])
    ed3 = jnp.stack([srcp.reshape(TOTC, CH), dstp.reshape(TOTC, CH)], axis=1)
    dst2 = dstp.reshape(NW, EPT)
    xp = jnp.pad(x, ((0, NPAD - N), (0, 0)))
    zeros = jnp.zeros((CH, D), jnp.float32)
    b1r = b1.reshape(1, D)
    b2r = b2.reshape(1, D)

    sc_degree, sc_scatter = _build_sc_kernels()
    hist = sc_degree(dst2)
    y1 = _tc_y1(hist, xp, W1)
    parts1 = sc_scatter(ed3, y1, zeros)
    z, y2 = _tc_layer1(hist, parts1, y1, b1r, W2)
    parts2 = sc_scatter(ed3, y2, zeros)
    out = _tc_layer2(hist, parts2, y2, b2r)
    return (z[:N], out[:N])
